# baseline (speedup 1.0000x reference)
import functools

import jax
import jax.numpy as jnp
from jax import lax
from jax.experimental import pallas as pl
from jax.experimental.pallas import tpu as pltpu
from jax.experimental.pallas import tpu_sc as plsc

D = 64
MARGIN = 1.0
LANES = 16
FCH = 160


@functools.cache
def _make_fuse(n_ent: int, n_rel: int):
    NC, NS = 2, 16
    NW = NC * NS

    mesh = plsc.VectorSubcoreMesh(core_axis_name="c", subcore_axis_name="s")

    @functools.partial(
        pl.kernel,
        mesh=mesh,
        compiler_params=pltpu.CompilerParams(
            needs_layout_passes=False, use_tc_tiling_on_sc=True),
        out_type=(jax.ShapeDtypeStruct((n_ent, 2 * D), jnp.float32),
                  jax.ShapeDtypeStruct((n_rel, 2 * D), jnp.float32)),
        scratch_types=[
            pltpu.VMEM((2, FCH, D), jnp.float32),
            pltpu.VMEM((2, FCH, D), jnp.float32),
            pltpu.VMEM((2, FCH, 2 * D), jnp.float32),
            pltpu.SemaphoreType.DMA,
            pltpu.SemaphoreType.DMA,
        ],
    )
    def k(ent_r, ent_i, rel_r, rel_i, entf, relf, bufa, bufb, buff,
          sem_r, sem_w):
        wid = lax.axis_index("s") * NC + lax.axis_index("c")

        def interleave(bi):
            def row(r, carry):
                for kk in range(D // LANES):
                    sl = pl.ds(kk * LANES, LANES)
                    sl2 = pl.ds(D + kk * LANES, LANES)
                    buff[bi, r, sl] = bufa[bi, r, sl]
                    buff[bi, r, sl2] = bufb[bi, r, sl]
                return carry

            lax.fori_loop(0, FCH, row, 0)

        def fuse_pair(a, b, dst, N):
            nchunks = N // FCH
            n_my = (nchunks - wid + NW - 1) // NW

            def reads(j, bi):
                r0 = (wid + j * NW) * FCH
                return (pltpu.async_copy(a.at[pl.ds(r0, FCH), :],
                                        bufa.at[bi], sem_r),
                        pltpu.async_copy(b.at[pl.ds(r0, FCH), :],
                                        bufb.at[bi], sem_r))

            def write(j, bi):
                r0 = (wid + j * NW) * FCH
                return pltpu.async_copy(buff.at[bi],
                                        dst.at[pl.ds(r0, FCH), :], sem_w)

            def body2(j2, carry):
                j0, j1 = j2 * 2, j2 * 2 + 1
                ra0, rb0 = reads(j0, 0)
                ra1, rb1 = reads(j1, 1)
                ra0.wait()
                rb0.wait()
                interleave(0)
                w0 = write(j0, 0)
                ra1.wait()
                rb1.wait()
                interleave(1)
                w1 = write(j1, 1)
                w0.wait()
                w1.wait()
                return carry

            lax.fori_loop(0, n_my // 2, body2, 0)

            @pl.when(n_my % 2 == 1)
            def _():
                j = n_my - 1
                ra, rb = reads(j, 0)
                ra.wait()
                rb.wait()
                interleave(0)
                write(j, 0).wait()

        fuse_pair(ent_r, ent_i, entf, n_ent)
        fuse_pair(rel_r, rel_i, relf, n_rel)

    return k


@functools.cache
def _make_score_kernel(B: int):
    NC, NS = 2, 16
    NW = NC * NS
    W = B // NW
    CH = 64
    NCHUNK = W // CH

    mesh = plsc.VectorSubcoreMesh(core_axis_name="c", subcore_axis_name="s")

    @functools.partial(
        pl.kernel,
        mesh=mesh,
        compiler_params=pltpu.CompilerParams(
            needs_layout_passes=False, use_tc_tiling_on_sc=False),
        out_type=jax.ShapeDtypeStruct((NW * LANES,), jnp.float32),
        scratch_types=[
            pltpu.VMEM((6, W), jnp.int32),
            pltpu.VMEM((6, CH, 2 * D), jnp.float32),
            pltpu.VMEM((LANES, LANES), jnp.float32),
            pltpu.VMEM((LANES,), jnp.float32),
            pltpu.SemaphoreType.DMA,
        ],
    )
    def k(ph, pr, pt, nh, nr, nt, entf, relf,
          out, idx_v, rows_v, part_v, res_v, sem):
        wid = lax.axis_index("s") * NC + lax.axis_index("c")
        base = wid * W

        for j, src in enumerate((ph, pr, pt, nh, nr, nt)):
            pltpu.sync_copy(src.at[pl.ds(base, W)], idx_v.at[j])

        plan = ((entf, 0), (relf, 1), (entf, 2),
                (entf, 3), (relf, 4), (entf, 5))

        def score(i, h_slot, r_slot, t_slot):
            p = jnp.zeros((LANES,), jnp.float32)
            for kk in range(D // LANES):
                re = pl.ds(kk * LANES, LANES)
                im = pl.ds(D + kk * LANES, LANES)
                hr = rows_v[h_slot, i, re]
                hi = rows_v[h_slot, i, im]
                rr = rows_v[r_slot, i, re]
                ri = rows_v[r_slot, i, im]
                tr = rows_v[t_slot, i, re]
                ti = rows_v[t_slot, i, im]
                p = p + rr * (hr * tr + hi * ti) + ri * (hr * ti - hi * tr)
            return p

        lane = lax.iota(jnp.int32, LANES)

        def pair_body(ii, g):
            i = g * LANES + ii
            part_v[ii, :] = score(i, 3, 4, 5) - score(i, 0, 1, 2)
            return g

        def group_body(g, acc):
            lax.fori_loop(0, LANES, pair_body, g)
            s = jnp.zeros((LANES,), jnp.float32)
            for j in range(LANES):
                s = s + plsc.load_gather(
                    part_v, [lane, jnp.full((LANES,), j, jnp.int32)])
            return acc + jnp.maximum(s + MARGIN, 0.0)

        def chunk_body(c, acc):
            copies = [
                pltpu.async_copy(
                    tbl.at[idx_v.at[jslot, pl.ds(c * CH, CH)]],
                    rows_v.at[slot], sem)
                for slot, (tbl, jslot) in enumerate(plan)
            ]
            for cp in copies:
                cp.wait()
            return lax.fori_loop(0, CH // LANES, group_body, acc)

        acc = lax.fori_loop(0, NCHUNK, chunk_body,
                            jnp.zeros((LANES,), jnp.float32))

        res_v[...] = acc
        pltpu.sync_copy(res_v, out.at[pl.ds(wid * LANES, LANES)])

    return k


def kernel(pos_exmpl, neg_exmpl, ent_real, ent_imag, rel_real, rel_imag):
    B = pos_exmpl.shape[1]
    n_ent, n_rel = ent_real.shape[0], rel_real.shape[0]
    entf, relf = _make_fuse(n_ent, n_rel)(ent_real, ent_imag,
                                          rel_real, rel_imag)
    k = _make_score_kernel(B)
    out = k(pos_exmpl[0], pos_exmpl[1], pos_exmpl[2],
            neg_exmpl[0], neg_exmpl[1], neg_exmpl[2],
            entf, relf)
    return jnp.sum(out)


# speedup vs baseline: 1.0006x; 1.0006x over previous
import functools

import jax
import jax.numpy as jnp
from jax import lax
from jax.experimental import pallas as pl
from jax.experimental.pallas import tpu as pltpu
from jax.experimental.pallas import tpu_sc as plsc

D = 64
MARGIN = 1.0
LANES = 16
FCH = 160


@functools.cache
def _make_fuse(n_ent: int, n_rel: int):
    NC, NS = 2, 16
    NW = NC * NS

    mesh = plsc.VectorSubcoreMesh(core_axis_name="c", subcore_axis_name="s")

    @functools.partial(
        pl.kernel,
        mesh=mesh,
        compiler_params=pltpu.CompilerParams(
            needs_layout_passes=False, use_tc_tiling_on_sc=True,
            skip_device_barrier=True),
        out_type=(jax.ShapeDtypeStruct((n_ent, 2 * D), jnp.float32),
                  jax.ShapeDtypeStruct((n_rel, 2 * D), jnp.float32)),
        scratch_types=[
            pltpu.VMEM((2, FCH, D), jnp.float32),
            pltpu.VMEM((2, FCH, D), jnp.float32),
            pltpu.VMEM((2, FCH, 2 * D), jnp.float32),
            pltpu.SemaphoreType.DMA,
            pltpu.SemaphoreType.DMA,
        ],
    )
    def k(ent_r, ent_i, rel_r, rel_i, entf, relf, bufa, bufb, buff,
          sem_r, sem_w):
        wid = lax.axis_index("s") * NC + lax.axis_index("c")

        def interleave(bi):
            def row(r, carry):
                for kk in range(D // LANES):
                    sl = pl.ds(kk * LANES, LANES)
                    sl2 = pl.ds(D + kk * LANES, LANES)
                    buff[bi, r, sl] = bufa[bi, r, sl]
                    buff[bi, r, sl2] = bufb[bi, r, sl]
                return carry

            lax.fori_loop(0, FCH, row, 0)

        def fuse_pair(a, b, dst, N):
            nchunks = N // FCH
            n_my = (nchunks - wid + NW - 1) // NW

            def reads(j, bi):
                r0 = (wid + j * NW) * FCH
                return (pltpu.async_copy(a.at[pl.ds(r0, FCH), :],
                                        bufa.at[bi], sem_r),
                        pltpu.async_copy(b.at[pl.ds(r0, FCH), :],
                                        bufb.at[bi], sem_r))

            def write(j, bi):
                r0 = (wid + j * NW) * FCH
                return pltpu.async_copy(buff.at[bi],
                                        dst.at[pl.ds(r0, FCH), :], sem_w)

            def body2(j2, carry):
                j0, j1 = j2 * 2, j2 * 2 + 1
                ra0, rb0 = reads(j0, 0)
                ra1, rb1 = reads(j1, 1)
                ra0.wait()
                rb0.wait()
                interleave(0)
                w0 = write(j0, 0)
                ra1.wait()
                rb1.wait()
                interleave(1)
                w1 = write(j1, 1)
                w0.wait()
                w1.wait()
                return carry

            lax.fori_loop(0, n_my // 2, body2, 0)

            @pl.when(n_my % 2 == 1)
            def _():
                j = n_my - 1
                ra, rb = reads(j, 0)
                ra.wait()
                rb.wait()
                interleave(0)
                write(j, 0).wait()

        fuse_pair(ent_r, ent_i, entf, n_ent)
        fuse_pair(rel_r, rel_i, relf, n_rel)

    return k


@functools.cache
def _make_score_kernel(B: int):
    NC, NS = 2, 16
    NW = NC * NS
    W = B // NW
    CH = 64
    NCHUNK = W // CH

    mesh = plsc.VectorSubcoreMesh(core_axis_name="c", subcore_axis_name="s")

    @functools.partial(
        pl.kernel,
        mesh=mesh,
        compiler_params=pltpu.CompilerParams(
            needs_layout_passes=False, use_tc_tiling_on_sc=False,
            skip_device_barrier=True),
        out_type=jax.ShapeDtypeStruct((NW * LANES,), jnp.float32),
        scratch_types=[
            pltpu.VMEM((6, W), jnp.int32),
            pltpu.VMEM((6, CH, 2 * D), jnp.float32),
            pltpu.VMEM((LANES, LANES), jnp.float32),
            pltpu.VMEM((LANES,), jnp.float32),
            pltpu.SemaphoreType.DMA,
        ],
    )
    def k(ph, pr, pt, nh, nr, nt, entf, relf,
          out, idx_v, rows_v, part_v, res_v, sem):
        wid = lax.axis_index("s") * NC + lax.axis_index("c")
        base = wid * W

        for j, src in enumerate((ph, pr, pt, nh, nr, nt)):
            pltpu.sync_copy(src.at[pl.ds(base, W)], idx_v.at[j])

        plan = ((entf, 0), (relf, 1), (entf, 2),
                (entf, 3), (relf, 4), (entf, 5))

        def score(i, h_slot, r_slot, t_slot):
            p = jnp.zeros((LANES,), jnp.float32)
            for kk in range(D // LANES):
                re = pl.ds(kk * LANES, LANES)
                im = pl.ds(D + kk * LANES, LANES)
                hr = rows_v[h_slot, i, re]
                hi = rows_v[h_slot, i, im]
                rr = rows_v[r_slot, i, re]
                ri = rows_v[r_slot, i, im]
                tr = rows_v[t_slot, i, re]
                ti = rows_v[t_slot, i, im]
                p = p + rr * (hr * tr + hi * ti) + ri * (hr * ti - hi * tr)
            return p

        lane = lax.iota(jnp.int32, LANES)

        def pair_body(ii, g):
            i = g * LANES + ii
            part_v[ii, :] = score(i, 3, 4, 5) - score(i, 0, 1, 2)
            return g

        def group_body(g, acc):
            lax.fori_loop(0, LANES, pair_body, g)
            s = jnp.zeros((LANES,), jnp.float32)
            for j in range(LANES):
                s = s + plsc.load_gather(
                    part_v, [lane, jnp.full((LANES,), j, jnp.int32)])
            return acc + jnp.maximum(s + MARGIN, 0.0)

        def chunk_body(c, acc):
            copies = [
                pltpu.async_copy(
                    tbl.at[idx_v.at[jslot, pl.ds(c * CH, CH)]],
                    rows_v.at[slot], sem)
                for slot, (tbl, jslot) in enumerate(plan)
            ]
            for cp in copies:
                cp.wait()
            return lax.fori_loop(0, CH // LANES, group_body, acc)

        acc = lax.fori_loop(0, NCHUNK, chunk_body,
                            jnp.zeros((LANES,), jnp.float32))

        res_v[...] = acc
        pltpu.sync_copy(res_v, out.at[pl.ds(wid * LANES, LANES)])

    return k


def kernel(pos_exmpl, neg_exmpl, ent_real, ent_imag, rel_real, rel_imag):
    B = pos_exmpl.shape[1]
    n_ent, n_rel = ent_real.shape[0], rel_real.shape[0]
    entf, relf = _make_fuse(n_ent, n_rel)(ent_real, ent_imag,
                                          rel_real, rel_imag)
    k = _make_score_kernel(B)
    out = k(pos_exmpl[0], pos_exmpl[1], pos_exmpl[2],
            neg_exmpl[0], neg_exmpl[1], neg_exmpl[2],
            entf, relf)
    return jnp.sum(out)


# speedup vs baseline: 1.1921x; 1.1914x over previous
import functools

import jax
import jax.numpy as jnp
from jax import lax
from jax.experimental import pallas as pl
from jax.experimental.pallas import tpu as pltpu
from jax.experimental.pallas import tpu_sc as plsc

D = 64
MARGIN = 1.0
LANES = 16


@functools.cache
def _make_kernel(B: int):
    NC, NS = 2, 16
    NW = NC * NS
    W = B // NW
    CH = 64
    NCHUNK = W // CH

    mesh = plsc.VectorSubcoreMesh(core_axis_name="c", subcore_axis_name="s")

    @functools.partial(
        pl.kernel,
        mesh=mesh,
        compiler_params=pltpu.CompilerParams(
            needs_layout_passes=False, use_tc_tiling_on_sc=False),
        out_type=jax.ShapeDtypeStruct((NW, LANES), jnp.float32),
        scratch_types=[
            pltpu.VMEM((6, W), jnp.int32),
            pltpu.VMEM((12, CH, D), jnp.float32),
            pltpu.VMEM((LANES, LANES), jnp.float32),
            pltpu.VMEM((LANES,), jnp.float32),
            pltpu.SemaphoreType.DMA,
        ],
    )
    def k(ph, pr, pt, nh, nr, nt, ent_r, ent_i, rel_r, rel_i,
          out, idx_v, rows_v, part_v, res_v, sem):
        wid = lax.axis_index("s") * NC + lax.axis_index("c")
        base = wid * W

        for j, src in enumerate((ph, pr, pt, nh, nr, nt)):
            pltpu.sync_copy(src.at[pl.ds(base, W)], idx_v.at[j])

        plan = ((ent_r, 0), (ent_i, 0), (rel_r, 1), (rel_i, 1),
                (ent_r, 2), (ent_i, 2),
                (ent_r, 3), (ent_i, 3), (rel_r, 4), (rel_i, 4),
                (ent_r, 5), (ent_i, 5))

        def score(i, h_slot, r_slot, t_slot):
            p = jnp.zeros((LANES,), jnp.float32)
            for kk in range(D // LANES):
                sl = pl.ds(kk * LANES, LANES)
                hr = rows_v[h_slot, i, sl]
                hi = rows_v[h_slot + 1, i, sl]
                rr = rows_v[r_slot, i, sl]
                ri = rows_v[r_slot + 1, i, sl]
                tr = rows_v[t_slot, i, sl]
                ti = rows_v[t_slot + 1, i, sl]
                p = p + rr * (hr * tr + hi * ti) + ri * (hr * ti - hi * tr)
            return p

        lane = lax.iota(jnp.int32, LANES)

        def pair_body(ii, g):
            i = g * LANES + ii
            part_v[ii, :] = score(i, 6, 8, 10) - score(i, 0, 2, 4)
            return g

        def group_body(g, acc):
            lax.fori_loop(0, LANES, pair_body, g)
            s = jnp.zeros((LANES,), jnp.float32)
            for j in range(LANES):
                s = s + plsc.load_gather(
                    part_v, [lane, jnp.full((LANES,), j, jnp.int32)])
            return acc + jnp.maximum(s + MARGIN, 0.0)

        def chunk_body(c, acc):
            copies = [
                pltpu.async_copy(
                    tbl.at[idx_v.at[jslot, pl.ds(c * CH, CH)]],
                    rows_v.at[slot], sem)
                for slot, (tbl, jslot) in enumerate(plan)
            ]
            for cp in copies:
                cp.wait()
            return lax.fori_loop(0, CH // LANES, group_body, acc)

        acc = lax.fori_loop(0, NCHUNK, chunk_body,
                            jnp.zeros((LANES,), jnp.float32))

        res_v[...] = acc
        pltpu.sync_copy(res_v, out.at[wid])

    return k


def kernel(pos_exmpl, neg_exmpl, ent_real, ent_imag, rel_real, rel_imag):
    B = pos_exmpl.shape[1]
    k = _make_kernel(B)
    out = k(pos_exmpl[0], pos_exmpl[1], pos_exmpl[2],
            neg_exmpl[0], neg_exmpl[1], neg_exmpl[2],
            ent_real, ent_imag, rel_real, rel_imag)
    return jnp.sum(out)


# speedup vs baseline: 1.4410x; 1.2088x over previous
import functools

import jax
import jax.numpy as jnp
from jax import lax
from jax.experimental import pallas as pl
from jax.experimental.pallas import tpu as pltpu
from jax.experimental.pallas import tpu_sc as plsc

D = 64
MARGIN = 1.0
LANES = 16


@functools.cache
def _make_score_kernel(B: int):
    NC, NS = 2, 16
    NW = NC * NS
    W = B // NW
    CH = 64
    NCHUNK = W // CH

    mesh = plsc.VectorSubcoreMesh(core_axis_name="c", subcore_axis_name="s")

    @functools.partial(
        pl.kernel,
        mesh=mesh,
        compiler_params=pltpu.CompilerParams(
            needs_layout_passes=False, use_tc_tiling_on_sc=False),
        out_type=jax.ShapeDtypeStruct((NW * LANES,), jnp.float32),
        scratch_types=[
            pltpu.VMEM((6, W), jnp.int32),
            pltpu.VMEM((6, CH, 2 * D), jnp.float32),
            pltpu.VMEM((LANES, LANES), jnp.float32),
            pltpu.VMEM((LANES,), jnp.float32),
            pltpu.SemaphoreType.DMA,
        ],
    )
    def k(ph, pr, pt, nh, nr, nt, entf, relf,
          out, idx_v, rows_v, part_v, res_v, sem):
        wid = lax.axis_index("s") * NC + lax.axis_index("c")
        base = wid * W

        for j, src in enumerate((ph, pr, pt, nh, nr, nt)):
            pltpu.sync_copy(src.at[pl.ds(base, W)], idx_v.at[j])

        plan = ((entf, 0), (relf, 1), (entf, 2),
                (entf, 3), (relf, 4), (entf, 5))

        def score(i, h_slot, r_slot, t_slot):
            p = jnp.zeros((LANES,), jnp.float32)
            for kk in range(D // LANES):
                re = pl.ds(kk * LANES, LANES)
                im = pl.ds(D + kk * LANES, LANES)
                hr = rows_v[h_slot, i, re]
                hi = rows_v[h_slot, i, im]
                rr = rows_v[r_slot, i, re]
                ri = rows_v[r_slot, i, im]
                tr = rows_v[t_slot, i, re]
                ti = rows_v[t_slot, i, im]
                p = p + rr * (hr * tr + hi * ti) + ri * (hr * ti - hi * tr)
            return p

        lane = lax.iota(jnp.int32, LANES)

        def pair_body(ii, g):
            i = g * LANES + ii
            part_v[ii, :] = score(i, 3, 4, 5) - score(i, 0, 1, 2)
            return g

        def group_body(g, acc):
            lax.fori_loop(0, LANES, pair_body, g)
            s = jnp.zeros((LANES,), jnp.float32)
            for j in range(LANES):
                s = s + plsc.load_gather(
                    part_v, [lane, jnp.full((LANES,), j, jnp.int32)])
            return acc + jnp.maximum(s + MARGIN, 0.0)

        def chunk_body(c, acc):
            copies = [
                pltpu.async_copy(
                    tbl.at[idx_v.at[jslot, pl.ds(c * CH, CH)]],
                    rows_v.at[slot], sem)
                for slot, (tbl, jslot) in enumerate(plan)
            ]
            for cp in copies:
                cp.wait()
            return lax.fori_loop(0, CH // LANES, group_body, acc)

        acc = lax.fori_loop(0, NCHUNK, chunk_body,
                            jnp.zeros((LANES,), jnp.float32))

        res_v[...] = acc
        pltpu.sync_copy(res_v, out.at[pl.ds(wid * LANES, LANES)])

    return k


def kernel(pos_exmpl, neg_exmpl, ent_real, ent_imag, rel_real, rel_imag):
    B = pos_exmpl.shape[1]
    entf = jnp.concatenate([ent_real, ent_imag], axis=1)
    relf = jnp.concatenate([rel_real, rel_imag], axis=1)
    k = _make_score_kernel(B)
    out = k(pos_exmpl[0], pos_exmpl[1], pos_exmpl[2],
            neg_exmpl[0], neg_exmpl[1], neg_exmpl[2],
            entf, relf)
    return jnp.sum(out)
